# speedup vs baseline: 44.1419x; 44.1419x over previous
import functools

import jax
import jax.numpy as jnp
from jax import lax
from jax.experimental import pallas as pl
from jax.experimental.pallas import tpu as pltpu

ALPHA = 0.5
NEG = -3.0e38


def _leaky(x):
    return jnp.where(x > 0, x, 0.2 * x)


def _prep_kernel(intent_ref, wk_ref, wq_ref, m_ref):
    k = lax.dot_general(intent_ref[...], wk_ref[...], (((1,), (1,)), ((), ())),
                        preferred_element_type=jnp.float32)
    m_ref[...] = lax.dot_general(k, wq_ref[...], (((1,), (0,)), ((), ())),
                                 preferred_element_type=jnp.float32)


def _pass1_kernel(item_ref, m_ref, intent_ref, wa_ref,
                  idx_ref, acc_ref, ss_ref):
    i = pl.program_id(0)
    blk = item_ref[...]
    B, d = blk.shape
    n_int = m_ref.shape[0]

    logits = lax.dot_general(blk, m_ref[...], (((1,), (1,)), ((), ())),
                             preferred_element_type=jnp.float32)
    iota = lax.broadcasted_iota(jnp.int32, (B, n_int), 1)

    def amax(l):
        m = jnp.max(l, axis=1, keepdims=True)
        return jnp.min(jnp.where(l == m, iota, n_int + 1), axis=1)

    i1 = amax(logits)
    l2 = jnp.where(iota == i1[:, None], NEG, logits)
    i2 = amax(l2)
    l3 = jnp.where(iota == i2[:, None], NEG, l2)
    i3 = amax(l3)

    smin = jnp.minimum(jnp.minimum(i1, i2), i3)
    smax = jnp.maximum(jnp.maximum(i1, i2), i3)
    smid = i1 + i2 + i3 - smin - smax

    oh1 = iota == smin[:, None]
    oh2 = iota == smid[:, None]
    oh3 = iota == smax[:, None]

    idx_ref[0, 0, :] = smin
    idx_ref[0, 1, :] = smid
    idx_ref[0, 2, :] = smax

    a1 = lax.dot_general(blk * wa_ref[...][None, :], intent_ref[...],
                         (((1,), (1,)), ((), ())),
                         preferred_element_type=jnp.float32)
    e1 = jnp.exp(_leaky(jnp.sum(jnp.where(oh1, a1, 0.0), axis=1)))
    e2 = jnp.exp(_leaky(jnp.sum(jnp.where(oh2, a1, 0.0), axis=1)))
    e3 = jnp.exp(_leaky(jnp.sum(jnp.where(oh3, a1, 0.0), axis=1)))

    P = (e1[:, None] * oh1.astype(jnp.float32)
         + e2[:, None] * oh2.astype(jnp.float32)
         + e3[:, None] * oh3.astype(jnp.float32))

    @pl.when(i == 0)
    def _init():
        acc_ref[...] = jnp.zeros_like(acc_ref)
        ss_ref[...] = jnp.zeros_like(ss_ref)

    acc_ref[...] += lax.dot_general(P, blk, (((0,), (0,)), ((), ())),
                                    preferred_element_type=jnp.float32)
    ss_ref[...] += jnp.sum(P, axis=0)


def _pass2_kernel(item_ref, idx_ref, acc_ref, ss_ref, wb_ref, intent_ref,
                  out_ref):
    blk = item_ref[...]
    B, d = blk.shape
    n_int = acc_ref.shape[0]

    ss = ss_ref[...]
    denom = jnp.where(ss == 0.0, 1.0, ss)
    intent_new = acc_ref[...] / denom[:, None]

    iota = lax.broadcasted_iota(jnp.int32, (B, n_int), 1)
    oh1 = iota == idx_ref[0, 0, :][:, None]
    oh2 = iota == idx_ref[0, 1, :][:, None]
    oh3 = iota == idx_ref[0, 2, :][:, None]

    a2 = lax.dot_general(blk * wb_ref[...][None, :], intent_new,
                         (((1,), (1,)), ((), ())),
                         preferred_element_type=jnp.float32)
    t1 = _leaky(jnp.sum(jnp.where(oh1, a2, 0.0), axis=1))
    t2 = _leaky(jnp.sum(jnp.where(oh2, a2, 0.0), axis=1))
    t3 = _leaky(jnp.sum(jnp.where(oh3, a2, 0.0), axis=1))
    f1 = jnp.exp(t1)
    f2 = jnp.exp(t2)
    f3 = jnp.exp(t3)
    s = f1 + f2 + f3
    w1 = f1 / s
    w2 = f2 / s
    w3 = f3 / s

    P2 = (w1[:, None] * oh1.astype(jnp.float32)
          + w2[:, None] * oh2.astype(jnp.float32)
          + w3[:, None] * oh3.astype(jnp.float32))
    nei = lax.dot_general(P2, intent_new, (((1,), (0,)), ((), ())),
                          preferred_element_type=jnp.float32)
    out_ref[...] = ALPHA * blk + (1.0 - ALPHA) * nei


def _pick_block(n):
    for b in range(2048, 0, -8):
        if n % b == 0:
            return b
    return n


def kernel(item_emb, n_items, intent_emb, n_intents, Wq, Wk, wa, wb):
    n, d = item_emb.shape
    n_int = intent_emb.shape[0]
    B = _pick_block(n)
    nb = n // B

    M = pl.pallas_call(
        _prep_kernel,
        out_shape=jax.ShapeDtypeStruct((n_int, d), jnp.float32),
    )(intent_emb, Wk, Wq)

    grid = (nb,)
    idx, acc, ss = pl.pallas_call(
        _pass1_kernel,
        grid=grid,
        in_specs=[
            pl.BlockSpec((B, d), lambda i: (i, 0)),
            pl.BlockSpec((n_int, d), lambda i: (0, 0)),
            pl.BlockSpec((n_int, d), lambda i: (0, 0)),
            pl.BlockSpec((d,), lambda i: (0,)),
        ],
        out_specs=[
            pl.BlockSpec((1, 3, B), lambda i: (i, 0, 0)),
            pl.BlockSpec((n_int, d), lambda i: (0, 0)),
            pl.BlockSpec((n_int,), lambda i: (0,)),
        ],
        out_shape=[
            jax.ShapeDtypeStruct((nb, 3, B), jnp.int32),
            jax.ShapeDtypeStruct((n_int, d), jnp.float32),
            jax.ShapeDtypeStruct((n_int,), jnp.float32),
        ],
        compiler_params=pltpu.CompilerParams(
            dimension_semantics=("arbitrary",)),
    )(item_emb, M, intent_emb, wa)

    out = pl.pallas_call(
        _pass2_kernel,
        grid=grid,
        in_specs=[
            pl.BlockSpec((B, d), lambda i: (i, 0)),
            pl.BlockSpec((1, 3, B), lambda i: (i, 0, 0)),
            pl.BlockSpec((n_int, d), lambda i: (0, 0)),
            pl.BlockSpec((n_int,), lambda i: (0,)),
            pl.BlockSpec((d,), lambda i: (0,)),
            pl.BlockSpec((n_int, d), lambda i: (0, 0)),
        ],
        out_specs=pl.BlockSpec((B, d), lambda i: (i, 0)),
        out_shape=jax.ShapeDtypeStruct((n, d), jnp.float32),
        compiler_params=pltpu.CompilerParams(
            dimension_semantics=("arbitrary",)),
    )(item_emb, idx, acc, ss, wb, intent_emb)
    return out
